# Initial kernel scaffold; baseline (speedup 1.0000x reference)
import jax
import jax.numpy as jnp
from jax.experimental import pallas as pl


def kernel(input, indices, pe):
    raise NotImplementedError("write your pallas kernel here")


# speedup vs baseline: 1.1602x; 1.1602x over previous
import functools

import jax
import jax.numpy as jnp
from jax import lax
from jax.experimental import pallas as pl
from jax.experimental.pallas import tpu as pltpu
from jax.experimental.pallas import tpu_sc as plsc

B, L, D, MAX_LEN = 4, 4096, 1024, 8192

NC, NS = 2, 16
NW = NC * NS
ROWS_PER_W = L // NW
CHUNK = 32
NCHUNK = ROWS_PER_W // CHUNK

_sc_mesh = plsc.VectorSubcoreMesh(core_axis_name="c", subcore_axis_name="s")


@functools.partial(
    pl.kernel,
    out_type=jax.ShapeDtypeStruct((L, D), jnp.float32),
    mesh=_sc_mesh,
    scratch_types=[
        pltpu.VMEM((NCHUNK, CHUNK), jnp.int32),
        pltpu.VMEM((2, CHUNK, D), jnp.float32),
        pltpu.SemaphoreType.DMA,
        pltpu.SemaphoreType.DMA,
    ],
)
def _sc_gather(pe_hbm, idx_hbm, out_hbm, idx_v, rows_v, sem_g, sem_s):
    wid = lax.axis_index("s") * NC + lax.axis_index("c")
    base = wid * ROWS_PER_W
    pltpu.sync_copy(idx_hbm.at[wid], idx_v)
    gathers = [None] * NCHUNK
    scatters = [None] * NCHUNK
    gathers[0] = pltpu.async_copy(pe_hbm.at[idx_v.at[0]], rows_v.at[0], sem_g)
    for c in range(NCHUNK):
        if c + 1 < NCHUNK:
            if c >= 1:
                scatters[c - 1].wait()
            gathers[c + 1] = pltpu.async_copy(
                pe_hbm.at[idx_v.at[c + 1]], rows_v.at[(c + 1) % 2], sem_g
            )
        gathers[c].wait()
        scatters[c] = pltpu.async_copy(
            rows_v.at[c % 2], out_hbm.at[pl.ds(base + c * CHUNK, CHUNK)], sem_s
        )
    scatters[NCHUNK - 2].wait()
    scatters[NCHUNK - 1].wait()


_LB = 256


def _add_body(in_ref, g_ref, out_ref):
    out_ref[...] = in_ref[...] + g_ref[...][None, :, :]


_tc_add = pl.pallas_call(
    _add_body,
    grid=(L // _LB,),
    in_specs=[
        pl.BlockSpec((B, _LB, D), lambda i: (0, i, 0)),
        pl.BlockSpec((_LB, D), lambda i: (i, 0)),
    ],
    out_specs=pl.BlockSpec((B, _LB, D), lambda i: (0, i, 0)),
    out_shape=jax.ShapeDtypeStruct((B, L, D), jnp.float32),
    compiler_params=pltpu.CompilerParams(
        dimension_semantics=("arbitrary",),
    ),
)


def kernel(input, indices, pe):
    idx = indices.astype(jnp.int32).reshape(NW, NCHUNK, CHUNK)
    gathered = _sc_gather(pe, idx)
    return _tc_add(input, gathered)
